# Initial kernel scaffold; baseline (speedup 1.0000x reference)
import jax
import jax.numpy as jnp
from jax.experimental import pallas as pl


def kernel(x, pos, batch, W1, b1, W2, b2, W3, b3):
    raise NotImplementedError("write your pallas kernel here")


# speedup vs baseline: 17.9722x; 17.9722x over previous
import functools

import jax
import jax.numpy as jnp
from jax import lax
from jax.experimental import pallas as pl
from jax.experimental.pallas import tpu as pltpu
from jax.experimental.pallas import tpu_sc as plsc

N = 10000
D = 128
K = 32
R2 = 0.25
H1 = 64
H2 = 64
H3 = 128

NW = 32
QPW = 320
NPAD = NW * QPW
NVP = 10016
BQ = 256
NBLK = NPAD // BQ

BIG = 3.0e38


def _k1_body(x_ref, pp_ref, w1x_ref, w1p_ref, b1_ref, a_ref, b_ref):
    p = jnp.dot(pp_ref[...], w1p_ref[...], preferred_element_type=jnp.float32)
    xw = jnp.dot(x_ref[...], w1x_ref[...], preferred_element_type=jnp.float32)
    z = jnp.zeros((BQ, D - H1), jnp.float32)
    a_ref[...] = jnp.concatenate([xw + p + b1_ref[...], z], axis=1)
    b_ref[...] = jnp.concatenate([-p, z], axis=1)


def _k1(xp, posp, w1x, w1pp, b1):
    return pl.pallas_call(
        _k1_body,
        grid=(NBLK,),
        in_specs=[
            pl.BlockSpec((BQ, D), lambda i: (i, 0)),
            pl.BlockSpec((BQ, D), lambda i: (i, 0)),
            pl.BlockSpec((D, H1), lambda i: (0, 0)),
            pl.BlockSpec((D, H1), lambda i: (0, 0)),
            pl.BlockSpec((1, H1), lambda i: (0, 0)),
        ],
        out_specs=[
            pl.BlockSpec((BQ, D), lambda i: (i, 0)),
            pl.BlockSpec((BQ, D), lambda i: (i, 0)),
        ],
        out_shape=[
            jax.ShapeDtypeStruct((NPAD, D), jnp.float32),
            jax.ShapeDtypeStruct((NPAD, D), jnp.float32),
        ],
    )(xp, posp, w1x, w1pp, b1)


def _merge_split(ak, ai, bk, bi):
    brk = lax.rev(bk, (0,))
    bri = lax.rev(bi, (0,))
    c = ak <= brk
    lok = jnp.where(c, ak, brk)
    loi = jnp.where(c, ai, bri)
    hik = jnp.where(c, brk, ak)
    hii = jnp.where(c, bri, ai)
    lok, loi = plsc.sort_key_val(lok, loi)
    hik, hii = plsc.sort_key_val(hik, hii)
    return lok, loi, hik, hii


def _graph_kernel(px_h, py_h, pz_h, bat_h, nbr_h, cnt_h,
                  px_v, py_v, pz_v, bat_v, buf_d, buf_i,
                  segs_v, sege_v, nbr_v, cnt_v):
    info = plsc.get_sparse_core_info()
    nc = info.num_cores
    wid = lax.axis_index("s") * nc + lax.axis_index("c")
    iota = lax.iota(jnp.int32, 16)

    pltpu.sync_copy(px_h, px_v)
    pltpu.sync_copy(py_h, py_v)
    pltpu.sync_copy(pz_h, pz_v)
    pltpu.sync_copy(bat_h, bat_v)

    zf = jnp.zeros((16,), jnp.float32)
    px_v[pl.ds(N, 16)] = zf
    py_v[pl.ds(N, 16)] = zf
    pz_v[pl.ds(N, 16)] = zf
    bat_v[pl.ds(N, 16)] = jnp.full((16,), -1, jnp.int32)
    segs_v[...] = jnp.zeros((16,), jnp.int32)
    sege_v[...] = jnp.zeros((16,), jnp.int32)

    def _seg_body(v, carry):
        j = v * 16 + iota
        bv = bat_v[pl.ds(v * 16, 16)]
        bprev = plsc.load_gather(bat_v, [jnp.maximum(j - 1, 0)])
        is_start = (j == 0) | (bv != bprev)
        plsc.store_scatter(segs_v, [bv], j, mask=is_start)
        bnext = plsc.load_gather(bat_v, [jnp.minimum(j + 1, N - 1)])
        is_end = (j == N - 1) | (bnext != bv)
        plsc.store_scatter(sege_v, [bv], j + 1, mask=is_end)
        return carry

    lax.fori_loop(0, N // 16, _seg_body, 0)

    def _query_body(ql, carry):
        q = wid * QPW + ql
        qe = jnp.minimum(q, N - 1)
        qsp = jnp.full((16,), qe, jnp.int32)
        qx = plsc.load_gather(px_v, [qsp])
        qy = plsc.load_gather(py_v, [qsp])
        qz = plsc.load_gather(pz_v, [qsp])
        bq = plsc.load_gather(bat_v, [qsp])
        s = jnp.max(plsc.load_gather(segs_v, [bq]))
        e = jnp.max(plsc.load_gather(sege_v, [bq]))

        def _scan_body(v, cnt):
            j = s + v * 16 + iota
            jc = jnp.minimum(j, NVP - 1)
            cx = plsc.load_gather(px_v, [jc])
            cy = plsc.load_gather(py_v, [jc])
            cz = plsc.load_gather(pz_v, [jc])
            dx = cx - qx
            dy = cy - qy
            dz = cz - qz
            d2 = dx * dx + dy * dy + dz * dz
            m = (d2 <= R2) & (j < e)
            inc = plsc.cumsum(jnp.where(m, 1, 0))
            pos = cnt + inc - 1
            plsc.store_scatter(buf_i, [pos], j, mask=m)
            plsc.store_scatter(buf_d, [pos], d2, mask=m)
            return cnt + jnp.max(inc)

        nvec = (e - s + 15) // 16
        cnt = lax.fori_loop(0, nvec, _scan_body, jnp.int32(0))

        def _small():
            c0 = buf_i[pl.ds(0, 16)]
            c1 = buf_i[pl.ds(16, 16)]
            return c0, c1, cnt

        def _big():
            infk = jnp.full((16,), BIG, jnp.float32)
            zi = jnp.zeros((16,), jnp.int32)

            def _mbody(c, carry):
                b0k, b0i, b1k, b1i = carry
                j = c * 16 + iota
                jc = jnp.minimum(j, NVP - 1)
                ck = plsc.load_gather(buf_d, [jc])
                ci = plsc.load_gather(buf_i, [jc])
                ck = jnp.where(j < cnt, ck, BIG)
                sk, si = plsc.sort_key_val(ck, ci)
                b0k, b0i, uhk, uhi = _merge_split(b0k, b0i, sk, si)
                b1k, b1i, _, _ = _merge_split(b1k, b1i, uhk, uhi)
                return b0k, b0i, b1k, b1i

            nch = (cnt + 15) // 16
            b0k, b0i, b1k, b1i = lax.fori_loop(
                0, nch, _mbody, (infk, zi, infk, zi))
            return b0i, b1i, jnp.int32(K)

        c0, c1, cnt_eff = lax.cond(cnt > K, _big, _small)
        sel0 = jnp.where(iota < cnt_eff, c0, qsp)
        sel1 = jnp.where(iota + 16 < cnt_eff, c1, qsp)
        qlsp = jnp.full((16,), ql, jnp.int32)
        plsc.store_scatter(nbr_v, [iota * QPW + qlsp], sel0)
        plsc.store_scatter(nbr_v, [(iota + 16) * QPW + qlsp], sel1)
        plsc.store_scatter(cnt_v, [qlsp],
                           jnp.full((16,), cnt_eff, jnp.int32),
                           mask=(iota == 0))
        return carry

    lax.fori_loop(0, QPW, _query_body, 0)

    pltpu.sync_copy(nbr_v, nbr_h.at[pl.ds(wid * K * QPW, K * QPW)])
    pltpu.sync_copy(cnt_v, cnt_h.at[pl.ds(wid * QPW, QPW)])


def _k2(posx, posy, posz, batv):
    mesh = plsc.VectorSubcoreMesh(core_axis_name="c", subcore_axis_name="s")
    kfn = functools.partial(
        pl.kernel,
        mesh=mesh,
        compiler_params=pltpu.CompilerParams(needs_layout_passes=False),
        out_type=[
            jax.ShapeDtypeStruct((K * NW * QPW,), jnp.int32),
            jax.ShapeDtypeStruct((NPAD,), jnp.int32),
        ],
        scratch_types=[
            pltpu.VMEM((NVP,), jnp.float32),
            pltpu.VMEM((NVP,), jnp.float32),
            pltpu.VMEM((NVP,), jnp.float32),
            pltpu.VMEM((NVP,), jnp.int32),
            pltpu.VMEM((NVP,), jnp.float32),
            pltpu.VMEM((NVP,), jnp.int32),
            pltpu.VMEM((16,), jnp.int32),
            pltpu.VMEM((16,), jnp.int32),
            pltpu.VMEM((K * QPW,), jnp.int32),
            pltpu.VMEM((QPW,), jnp.int32),
        ],
    )(_graph_kernel)
    return kfn(posx, posy, posz, batv)


CH = 64
NCH = QPW // CH


def _gather_kernel(a_h, nbr_h, g_h, idx_v, rows_v, gsem, ssem):
    info = plsc.get_sparse_core_info()
    nc = info.num_cores
    wid = lax.axis_index("s") * nc + lax.axis_index("c")

    nsteps = K * NCH
    gh = [None] * 4
    sh = [None] * 4
    for st in range(nsteps):
        b = st % 4
        k = st // NCH
        c = st % NCH
        if sh[b] is not None:
            sh[b].wait()
        idx_chunk = idx_v.at[b, pl.ds(0, CH)]
        pltpu.sync_copy(
            nbr_h.at[pl.ds(wid * K * QPW + k * QPW + c * CH, CH)],
            idx_chunk)
        gh[b] = pltpu.async_copy(a_h.at[idx_chunk], rows_v.at[b], gsem[b])
        if st >= 1:
            b1 = (st - 1) % 4
            k1 = (st - 1) // NCH
            c1 = (st - 1) % NCH
            gh[b1].wait()
            sh[b1] = pltpu.async_copy(
                rows_v.at[b1],
                g_h.at[k1, pl.ds(wid * QPW + c1 * CH, CH)],
                ssem[b1])
    bl = (nsteps - 1) % 4
    gh[bl].wait()
    sh[bl] = pltpu.async_copy(
        rows_v.at[bl],
        g_h.at[K - 1, pl.ds(wid * QPW + (NCH - 1) * CH, CH)],
        ssem[bl])
    for b in range(4):
        if sh[b] is not None:
            sh[b].wait()


def _k3(a, nbr):
    mesh = plsc.VectorSubcoreMesh(core_axis_name="c", subcore_axis_name="s")
    kfn = functools.partial(
        pl.kernel,
        mesh=mesh,
        compiler_params=pltpu.CompilerParams(needs_layout_passes=False),
        out_type=jax.ShapeDtypeStruct((K, NPAD, D), jnp.float32),
        scratch_types=[
            pltpu.VMEM((4, 128), jnp.int32),
            pltpu.VMEM((4, CH, D), jnp.float32),
            [pltpu.SemaphoreType.DMA] * 4,
            [pltpu.SemaphoreType.DMA] * 4,
        ],
    )(_gather_kernel)
    return kfn(a, nbr)


def _k4_body(g_ref, b_ref, cnt_ref, w2_ref, b2_ref, w3_ref, b3_ref, out_ref):
    cnt = cnt_ref[...]
    bmat = b_ref[...]
    w2 = w2_ref[...]
    b2 = b2_ref[...]
    w3 = w3_ref[...]
    b3 = b3_ref[...]

    def body(k, acc):
        gk = g_ref[k]
        h1 = jnp.maximum(gk + bmat, 0.0)
        h2 = jnp.maximum(
            jnp.dot(h1, w2, preferred_element_type=jnp.float32) + b2, 0.0)
        h3 = jnp.dot(h2, w3, preferred_element_type=jnp.float32) + b3
        h3 = jnp.where(k < cnt, h3, -1e30)
        return jnp.maximum(acc, h3)

    acc = lax.fori_loop(0, K, body, jnp.full((BQ, H3), -1e30, jnp.float32))
    out_ref[...] = jnp.where(acc <= -1e29, 0.0, acc)


def _k4(g, bm, cnt2, w2, b2, w3, b3):
    return pl.pallas_call(
        _k4_body,
        grid=(NBLK,),
        in_specs=[
            pl.BlockSpec((K, BQ, D), lambda i: (0, i, 0)),
            pl.BlockSpec((BQ, D), lambda i: (i, 0)),
            pl.BlockSpec((BQ, 1), lambda i: (i, 0)),
            pl.BlockSpec((D, H2), lambda i: (0, 0)),
            pl.BlockSpec((1, H2), lambda i: (0, 0)),
            pl.BlockSpec((H2, H3), lambda i: (0, 0)),
            pl.BlockSpec((1, H3), lambda i: (0, 0)),
        ],
        out_specs=pl.BlockSpec((BQ, H3), lambda i: (i, 0)),
        out_shape=jax.ShapeDtypeStruct((NPAD, H3), jnp.float32),
    )(g, bm, cnt2, w2, b2, w3, b3)


def kernel(x, pos, batch, W1, b1, W2, b2, W3, b3):
    xp = jnp.pad(x, ((0, NPAD - N), (0, 0)))
    posp = jnp.pad(pos, ((0, NPAD - N), (0, D - 3)))
    w1x = W1[:D]
    w1pp = jnp.pad(W1[D:], ((0, D - 3), (0, 0)))

    a, bm = _k1(xp, posp, w1x, w1pp, b1.reshape(1, H1))

    posx = jnp.pad(pos[:, 0], (0, NVP - N))
    posy = jnp.pad(pos[:, 1], (0, NVP - N))
    posz = jnp.pad(pos[:, 2], (0, NVP - N))
    batv = jnp.pad(batch, (0, NVP - N))
    nbr, cnt = _k2(posx, posy, posz, batv)

    g = _k3(a, nbr)

    w2p = jnp.pad(W2, ((0, D - H1), (0, 0)))
    out = _k4(g, bm, cnt.reshape(NPAD, 1), w2p, b2.reshape(1, H2),
              W3, b3.reshape(1, H3))
    return (out[:N], pos, batch)


# speedup vs baseline: 21.7039x; 1.2076x over previous
import functools

import jax
import jax.numpy as jnp
from jax import lax
from jax.experimental import pallas as pl
from jax.experimental.pallas import tpu as pltpu
from jax.experimental.pallas import tpu_sc as plsc

N = 10000
D = 128
K = 32
R2 = 0.25
H1 = 64
H2 = 64
H3 = 128

NW = 32
QPW = 320
NPAD = NW * QPW
NVP = 10016
BQ = 256
NBLK = NPAD // BQ

BIG = 3.0e38


def _k1_body(x_ref, pp_ref, w1x_ref, w1p_ref, b1_ref, a_ref, b_ref):
    p = jnp.dot(pp_ref[...], w1p_ref[...], preferred_element_type=jnp.float32)
    xw = jnp.dot(x_ref[...], w1x_ref[...], preferred_element_type=jnp.float32)
    a = xw + p + b1_ref[...]
    a_ref[...] = jnp.concatenate(
        [a, jnp.zeros((BQ, D - H1), jnp.float32)], axis=1)
    b_ref[...] = -p


def _k1(xp, posp, w1x, w1pp, b1):
    return pl.pallas_call(
        _k1_body,
        grid=(NBLK,),
        in_specs=[
            pl.BlockSpec((BQ, D), lambda i: (i, 0)),
            pl.BlockSpec((BQ, D), lambda i: (i, 0)),
            pl.BlockSpec((D, H1), lambda i: (0, 0)),
            pl.BlockSpec((D, H1), lambda i: (0, 0)),
            pl.BlockSpec((1, H1), lambda i: (0, 0)),
        ],
        out_specs=[
            pl.BlockSpec((BQ, D), lambda i: (i, 0)),
            pl.BlockSpec((BQ, H1), lambda i: (i, 0)),
        ],
        out_shape=[
            jax.ShapeDtypeStruct((NPAD, D), jnp.float32),
            jax.ShapeDtypeStruct((NPAD, H1), jnp.float32),
        ],
    )(xp, posp, w1x, w1pp, b1)


def _merge_split(ak, ai, bk, bi):
    brk = lax.rev(bk, (0,))
    bri = lax.rev(bi, (0,))
    c = ak <= brk
    lok = jnp.where(c, ak, brk)
    loi = jnp.where(c, ai, bri)
    hik = jnp.where(c, brk, ak)
    hii = jnp.where(c, bri, ai)
    lok, loi = plsc.sort_key_val(lok, loi)
    hik, hii = plsc.sort_key_val(hik, hii)
    return lok, loi, hik, hii


def _graph_kernel(px_h, py_h, pz_h, bat_h, nbr_h, cnt_h,
                  px_v, py_v, pz_v, bat_v, buf_d, buf_i,
                  segs_v, sege_v, nbr_v, cnt_v):
    info = plsc.get_sparse_core_info()
    nc = info.num_cores
    wid = lax.axis_index("s") * nc + lax.axis_index("c")
    iota = lax.iota(jnp.int32, 16)

    pltpu.sync_copy(px_h, px_v)
    pltpu.sync_copy(py_h, py_v)
    pltpu.sync_copy(pz_h, pz_v)
    pltpu.sync_copy(bat_h, bat_v)

    zf = jnp.zeros((16,), jnp.float32)
    px_v[pl.ds(N, 16)] = zf
    py_v[pl.ds(N, 16)] = zf
    pz_v[pl.ds(N, 16)] = zf
    bat_v[pl.ds(N, 16)] = jnp.full((16,), -1, jnp.int32)
    segs_v[...] = jnp.zeros((16,), jnp.int32)
    sege_v[...] = jnp.zeros((16,), jnp.int32)

    def _seg_body(v, carry):
        j = v * 16 + iota
        bv = bat_v[pl.ds(v * 16, 16)]
        bprev = plsc.load_gather(bat_v, [jnp.maximum(j - 1, 0)])
        is_start = (j == 0) | (bv != bprev)
        plsc.store_scatter(segs_v, [bv], j, mask=is_start)
        bnext = plsc.load_gather(bat_v, [jnp.minimum(j + 1, N - 1)])
        is_end = (j == N - 1) | (bnext != bv)
        plsc.store_scatter(sege_v, [bv], j + 1, mask=is_end)
        return carry

    lax.fori_loop(0, N // 16, _seg_body, 0)

    def _query_body(ql, carry):
        q = wid * QPW + ql
        qe = jnp.minimum(q, N - 1)
        qsp = jnp.full((16,), qe, jnp.int32)
        qx = plsc.load_gather(px_v, [qsp])
        qy = plsc.load_gather(py_v, [qsp])
        qz = plsc.load_gather(pz_v, [qsp])
        bq = plsc.load_gather(bat_v, [qsp])
        s = jnp.max(plsc.load_gather(segs_v, [bq]))
        e = jnp.max(plsc.load_gather(sege_v, [bq]))

        def _scan_body(v, cnt):
            j = s + v * 16 + iota
            jc = jnp.minimum(j, NVP - 1)
            cx = plsc.load_gather(px_v, [jc])
            cy = plsc.load_gather(py_v, [jc])
            cz = plsc.load_gather(pz_v, [jc])
            dx = cx - qx
            dy = cy - qy
            dz = cz - qz
            d2 = dx * dx + dy * dy + dz * dz
            m = (d2 <= R2) & (j < e)
            inc = plsc.cumsum(jnp.where(m, 1, 0))
            pos = cnt + inc - 1
            plsc.store_scatter(buf_i, [pos], j, mask=m)
            plsc.store_scatter(buf_d, [pos], d2, mask=m)
            return cnt + jnp.max(inc)

        nvec = (e - s + 15) // 16
        cnt = lax.fori_loop(0, nvec, _scan_body, jnp.int32(0))

        def _small():
            c0 = buf_i[pl.ds(0, 16)]
            c1 = buf_i[pl.ds(16, 16)]
            return c0, c1, cnt

        def _big():
            infk = jnp.full((16,), BIG, jnp.float32)
            zi = jnp.zeros((16,), jnp.int32)

            def _mbody(c, carry):
                b0k, b0i, b1k, b1i = carry
                j = c * 16 + iota
                jc = jnp.minimum(j, NVP - 1)
                ck = plsc.load_gather(buf_d, [jc])
                ci = plsc.load_gather(buf_i, [jc])
                ck = jnp.where(j < cnt, ck, BIG)
                sk, si = plsc.sort_key_val(ck, ci)
                b0k, b0i, uhk, uhi = _merge_split(b0k, b0i, sk, si)
                b1k, b1i, _, _ = _merge_split(b1k, b1i, uhk, uhi)
                return b0k, b0i, b1k, b1i

            nch = (cnt + 15) // 16
            b0k, b0i, b1k, b1i = lax.fori_loop(
                0, nch, _mbody, (infk, zi, infk, zi))
            return b0i, b1i, jnp.int32(K)

        c0, c1, cnt_eff = lax.cond(cnt > K, _big, _small)
        sel0 = jnp.where(iota < cnt_eff, c0, qsp)
        sel1 = jnp.where(iota + 16 < cnt_eff, c1, qsp)
        qlsp = jnp.full((16,), ql, jnp.int32)
        plsc.store_scatter(nbr_v, [iota * QPW + qlsp], sel0)
        plsc.store_scatter(nbr_v, [(iota + 16) * QPW + qlsp], sel1)
        plsc.store_scatter(cnt_v, [qlsp],
                           jnp.full((16,), cnt_eff, jnp.int32),
                           mask=(iota == 0))
        return carry

    lax.fori_loop(0, QPW, _query_body, 0)

    pltpu.sync_copy(nbr_v, nbr_h.at[pl.ds(wid * K * QPW, K * QPW)])
    pltpu.sync_copy(cnt_v, cnt_h.at[pl.ds(wid * QPW, QPW)])


def _k2(posx, posy, posz, batv):
    mesh = plsc.VectorSubcoreMesh(core_axis_name="c", subcore_axis_name="s")
    kfn = functools.partial(
        pl.kernel,
        mesh=mesh,
        compiler_params=pltpu.CompilerParams(needs_layout_passes=False),
        out_type=[
            jax.ShapeDtypeStruct((K * NW * QPW,), jnp.int32),
            jax.ShapeDtypeStruct((NPAD,), jnp.int32),
        ],
        scratch_types=[
            pltpu.VMEM((NVP,), jnp.float32),
            pltpu.VMEM((NVP,), jnp.float32),
            pltpu.VMEM((NVP,), jnp.float32),
            pltpu.VMEM((NVP,), jnp.int32),
            pltpu.VMEM((NVP,), jnp.float32),
            pltpu.VMEM((NVP,), jnp.int32),
            pltpu.VMEM((16,), jnp.int32),
            pltpu.VMEM((16,), jnp.int32),
            pltpu.VMEM((K * QPW,), jnp.int32),
            pltpu.VMEM((QPW,), jnp.int32),
        ],
    )(_graph_kernel)
    return kfn(posx, posy, posz, batv)


CH = 64
NCH = QPW // CH


def _gather_kernel(a_h, nbr_h, g_h, idx_v, rows_v, gsem, ssem):
    info = plsc.get_sparse_core_info()
    nc = info.num_cores
    wid = lax.axis_index("s") * nc + lax.axis_index("c")

    nsteps = K * NCH
    gh = [None] * 4
    sh = [None] * 4
    for st in range(nsteps):
        b = st % 4
        k = st // NCH
        c = st % NCH
        if sh[b] is not None:
            sh[b].wait()
        idx_chunk = idx_v.at[b, pl.ds(0, CH)]
        pltpu.sync_copy(
            nbr_h.at[pl.ds(wid * K * QPW + k * QPW + c * CH, CH)],
            idx_chunk)
        gh[b] = pltpu.async_copy(a_h.at[idx_chunk], rows_v.at[b], gsem[b])
        if st >= 1:
            b1 = (st - 1) % 4
            k1 = (st - 1) // NCH
            c1 = (st - 1) % NCH
            gh[b1].wait()
            sh[b1] = pltpu.async_copy(
                rows_v.at[b1],
                g_h.at[k1, pl.ds(wid * QPW + c1 * CH, CH)],
                ssem[b1])
    bl = (nsteps - 1) % 4
    gh[bl].wait()
    sh[bl] = pltpu.async_copy(
        rows_v.at[bl],
        g_h.at[K - 1, pl.ds(wid * QPW + (NCH - 1) * CH, CH)],
        ssem[bl])
    for b in range(4):
        if sh[b] is not None:
            sh[b].wait()


def _k3(a, nbr):
    mesh = plsc.VectorSubcoreMesh(core_axis_name="c", subcore_axis_name="s")
    kfn = functools.partial(
        pl.kernel,
        mesh=mesh,
        compiler_params=pltpu.CompilerParams(needs_layout_passes=False),
        out_type=jax.ShapeDtypeStruct((K, NPAD, D), jnp.float32),
        scratch_types=[
            pltpu.VMEM((4, 128), jnp.int32),
            pltpu.VMEM((4, CH, D), jnp.float32),
            [pltpu.SemaphoreType.DMA] * 4,
            [pltpu.SemaphoreType.DMA] * 4,
        ],
    )(_gather_kernel)
    return kfn(a, nbr)


def _k4_body(g_ref, b_ref, cnt_ref, w2_ref, b2_ref, w3_ref, b3_ref, out_ref):
    cnt = cnt_ref[...]
    bmat = b_ref[...]

    g2 = g_ref[:, :, :H1].reshape(K * BQ, H1)
    bexp = jnp.broadcast_to(bmat[None], (K, BQ, H1)).reshape(K * BQ, H1)
    h1 = jnp.maximum(g2 + bexp, 0.0)
    h2 = jnp.maximum(
        jnp.dot(h1, w2_ref[...], preferred_element_type=jnp.float32)
        + b2_ref[...], 0.0)
    h3 = (jnp.dot(h2, w3_ref[...], preferred_element_type=jnp.float32)
          + b3_ref[...])
    h3r = h3.reshape(K, BQ, H3)
    kio = lax.broadcasted_iota(jnp.int32, (K, BQ, 1), 0)
    h3m = jnp.where(kio < cnt[None], h3r, -1e30)
    acc = jnp.max(h3m, axis=0)
    out_ref[...] = jnp.where(acc <= -1e29, 0.0, acc)


def _k4(g, bm, cnt2, w2, b2, w3, b3):
    return pl.pallas_call(
        _k4_body,
        grid=(NBLK,),
        in_specs=[
            pl.BlockSpec((K, BQ, D), lambda i: (0, i, 0)),
            pl.BlockSpec((BQ, H1), lambda i: (i, 0)),
            pl.BlockSpec((BQ, 1), lambda i: (i, 0)),
            pl.BlockSpec((H1, H2), lambda i: (0, 0)),
            pl.BlockSpec((1, H2), lambda i: (0, 0)),
            pl.BlockSpec((H2, H3), lambda i: (0, 0)),
            pl.BlockSpec((1, H3), lambda i: (0, 0)),
        ],
        out_specs=pl.BlockSpec((BQ, H3), lambda i: (i, 0)),
        out_shape=jax.ShapeDtypeStruct((NPAD, H3), jnp.float32),
    )(g, bm, cnt2, w2, b2, w3, b3)


def kernel(x, pos, batch, W1, b1, W2, b2, W3, b3):
    xp = jnp.pad(x, ((0, NPAD - N), (0, 0)))
    posp = jnp.pad(pos, ((0, NPAD - N), (0, D - 3)))
    w1x = W1[:D]
    w1pp = jnp.pad(W1[D:], ((0, D - 3), (0, 0)))

    a, bm = _k1(xp, posp, w1x, w1pp, b1.reshape(1, H1))

    posx = jnp.pad(pos[:, 0], (0, NVP - N))
    posy = jnp.pad(pos[:, 1], (0, NVP - N))
    posz = jnp.pad(pos[:, 2], (0, NVP - N))
    batv = jnp.pad(batch, (0, NVP - N))
    nbr, cnt = _k2(posx, posy, posz, batv)

    g = _k3(a, nbr)

    out = _k4(g, bm, cnt.reshape(NPAD, 1), W2, b2.reshape(1, H2),
              W3, b3.reshape(1, H3))
    return (out[:N], pos, batch)
